# Initial kernel scaffold; baseline (speedup 1.0000x reference)
import jax
import jax.numpy as jnp
from jax.experimental import pallas as pl


def kernel(x, edge_index, edge_attr, batch, emb_table, W_rel, W_root, bias):
    raise NotImplementedError("write your pallas kernel here")


# speedup vs baseline: 49.1273x; 49.1273x over previous
import functools

import jax
import jax.numpy as jnp
from jax import lax
from jax.experimental import pallas as pl
from jax.experimental.pallas import tpu as pltpu
from jax.experimental.pallas import tpu_sc as plsc

_N = 10000
_E = 320000
_G = 256
_R = 16
_V = 33
_DO = 64
_SEG = _N * _R
_AB = _R * _G * _V
_CB = _G * _V
_EPT = _E // 32
_EPS = _E // 16
_NPT = 313


def _loop16(n, body):
    def b(i, c):
        body(i)
        return c
    lax.fori_loop(0, n, b, 0)


def _sc_body(dst_h, src_h, rel_h, x_h, b_h, out_a, out_c,
             e_i1, e_i2, e_i3, segb, fones, fval, xtab, btab, nbuck, nval,
             cnt_sh, a_sh, c_sh):
    cid = lax.axis_index("c")
    sid = lax.axis_index("s")
    wid = cid * 16 + sid
    iota = lax.iota(jnp.int32, 16)

    def init_i(i):
        fones[pl.ds(i * 16, 16)] = jnp.full((16,), 1.0, jnp.float32)
        fval[pl.ds(i * 16, 16)] = jnp.zeros((16,), jnp.float32)
    _loop16(625, init_i)

    pltpu.sync_copy(fval.at[pl.ds(0, _SEG // 16)],
                    cnt_sh.at[pl.ds(sid * (_SEG // 16), _SEG // 16)])
    pltpu.sync_copy(fval.at[pl.ds(0, _AB // 16)],
                    a_sh.at[pl.ds(sid * (_AB // 16), _AB // 16)])
    pltpu.sync_copy(fval.at[pl.ds(0, _CB // 16)],
                    c_sh.at[pl.ds(sid * (_CB // 16), _CB // 16)])
    pltpu.sync_copy(x_h, xtab)
    pltpu.sync_copy(b_h, btab)
    plsc.subcore_barrier()

    for k in range(2):
        base = sid * _EPS + k * 10000
        pltpu.sync_copy(dst_h.at[pl.ds(base, 10000)], e_i1)
        pltpu.sync_copy(rel_h.at[pl.ds(base, 10000)], e_i2)

        def seg_i(i):
            sl = pl.ds(i * 16, 16)
            segb[sl] = e_i1[sl] * 16 + e_i2[sl]
        _loop16(625, seg_i)
        pltpu.sync_copy(fones, cnt_sh.at[segb], add=True)
    plsc.subcore_barrier()

    base = wid * _EPT
    pltpu.sync_copy(dst_h.at[pl.ds(base, _EPT)], e_i1)
    pltpu.sync_copy(rel_h.at[pl.ds(base, _EPT)], e_i2)
    pltpu.sync_copy(src_h.at[pl.ds(base, _EPT)], e_i3)

    def seg2_i(i):
        sl = pl.ds(i * 16, 16)
        segb[sl] = e_i1[sl] * 16 + e_i2[sl]
    _loop16(625, seg2_i)
    pltpu.sync_copy(cnt_sh.at[segb], fval)

    def buck_i(i):
        sl = pl.ds(i * 16, 16)
        g16 = plsc.load_gather(btab, [e_i1[sl]])
        v16 = plsc.load_gather(xtab, [e_i3[sl]])
        segb[sl] = e_i2[sl] * _CB + g16 * _V + v16
        fval[sl] = 1.0 / fval[sl]
    _loop16(625, buck_i)
    pltpu.sync_copy(fval, a_sh.at[segb], add=True)

    nb = wid * _NPT
    lim = jnp.minimum(nb + _NPT, _N)

    def node_i(i):
        idx16 = nb + i * 16 + iota
        ok = idx16 < lim
        cidx = jnp.minimum(idx16, _N - 1)
        g16 = plsc.load_gather(btab, [cidx])
        v16 = plsc.load_gather(xtab, [cidx])
        nbuck[pl.ds(i * 16, 16)] = g16 * _V + v16
        nval[pl.ds(i * 16, 16)] = jnp.where(ok, 1.0, 0.0)
    _loop16(20, node_i)
    pltpu.sync_copy(nval, c_sh.at[nbuck], add=True)
    plsc.subcore_barrier()

    pltpu.sync_copy(a_sh.at[pl.ds(sid * (_AB // 16), _AB // 16)],
                    fval.at[pl.ds(0, _AB // 16)])
    pltpu.sync_copy(fval.at[pl.ds(0, _AB // 16)],
                    out_a.at[pl.ds(cid * _AB + sid * (_AB // 16), _AB // 16)])
    pltpu.sync_copy(c_sh.at[pl.ds(sid * (_CB // 16), _CB // 16)],
                    fones.at[pl.ds(0, _CB // 16)])
    pltpu.sync_copy(fones.at[pl.ds(0, _CB // 16)],
                    out_c.at[pl.ds(cid * _CB + sid * (_CB // 16), _CB // 16)])


@functools.lru_cache(maxsize=1)
def _get_sc_call():
    return pl.kernel(
        _sc_body,
        out_type=(jax.ShapeDtypeStruct((2 * _AB,), jnp.float32),
                  jax.ShapeDtypeStruct((2 * _CB,), jnp.float32)),
        mesh=plsc.VectorSubcoreMesh(core_axis_name="c", subcore_axis_name="s"),
        compiler_params=pltpu.CompilerParams(needs_layout_passes=False),
        scratch_types=[
            pltpu.VMEM((10000,), jnp.int32),
            pltpu.VMEM((10000,), jnp.int32),
            pltpu.VMEM((10000,), jnp.int32),
            pltpu.VMEM((10000,), jnp.int32),
            pltpu.VMEM((10000,), jnp.float32),
            pltpu.VMEM((10000,), jnp.float32),
            pltpu.VMEM((10000,), jnp.int32),
            pltpu.VMEM((10000,), jnp.int32),
            pltpu.VMEM((320,), jnp.int32),
            pltpu.VMEM((320,), jnp.float32),
            pltpu.VMEM_SHARED((_SEG,), jnp.float32),
            pltpu.VMEM_SHARED((_AB,), jnp.float32),
            pltpu.VMEM_SHARED((_CB,), jnp.float32),
        ],
    )


def _tc_body(a_ref, c_ref, emb_ref, wrel_ref, wroot_ref, bias_ref, out_ref):
    hi = jax.lax.Precision.HIGHEST
    emb = emb_ref[...]
    c0 = c_ref[0] + c_ref[1]
    acc = jnp.zeros((_G, _DO), jnp.float32)
    for r in range(_R):
        t_r = jax.lax.dot(emb, wrel_ref[r], precision=hi)
        a_r = a_ref[0, r] + a_ref[1, r]
        acc = acc + jax.lax.dot(a_r, t_r, precision=hi)
    t_root = jax.lax.dot(emb, wroot_ref[...], precision=hi)
    acc = acc + jax.lax.dot(c0, t_root, precision=hi)
    cg = jnp.sum(c0, axis=1, keepdims=True)
    acc = acc + cg * bias_ref[...]
    out_ref[...] = acc / jnp.maximum(cg, 1.0)


_tc_call = pl.pallas_call(
    _tc_body,
    out_shape=jax.ShapeDtypeStruct((_G, _DO), jnp.float32),
)


@jax.jit
def kernel(x, edge_index, edge_attr, batch, emb_table, W_rel, W_root, bias):
    src = edge_index[0].astype(jnp.int32)
    dst = edge_index[1].astype(jnp.int32)
    rel = edge_attr.reshape(-1).astype(jnp.int32)
    xf = x.reshape(-1).astype(jnp.int32)
    bt = batch.astype(jnp.int32)
    out_a, out_c = _get_sc_call()(dst, src, rel, xf, bt)
    a4 = out_a.reshape(2, _R, _G, _V)
    c3 = out_c.reshape(2, _G, _V)
    return _tc_call(a4, c3, emb_table, W_rel, W_root, bias.reshape(1, _DO))


# speedup vs baseline: 54.0063x; 1.0993x over previous
import functools

import jax
import jax.numpy as jnp
from jax import lax
from jax.experimental import pallas as pl
from jax.experimental.pallas import tpu as pltpu
from jax.experimental.pallas import tpu_sc as plsc

_N = 10000
_E = 320000
_G = 256
_R = 16
_V = 33
_DO = 64
_SEG = _N * _R
_AB = _R * _G * _V
_CB = _G * _V
_EPT = _E // 32
_NPT = 313


def _loop16(n, body, unroll=8):
    def b(i, c):
        body(i)
        return c
    lax.fori_loop(0, n, b, 0, unroll=unroll)


def _sc_body(ei_h, ea_h, x_h, b_h, out_a, out_c,
             e_i1, e_i2, e_i3, segb, fones, fval, xtab, btab, nbuck, nval,
             cnt_sh, a_sh, c_sh):
    cid = lax.axis_index("c")
    sid = lax.axis_index("s")
    wid = cid * 16 + sid
    iota = lax.iota(jnp.int32, 16)

    def init_i(i):
        fones[pl.ds(i * 16, 16)] = jnp.full((16,), 1.0, jnp.float32)
        fval[pl.ds(i * 16, 16)] = jnp.zeros((16,), jnp.float32)
    _loop16(625, init_i)

    pltpu.sync_copy(fval.at[pl.ds(0, _SEG // 16)],
                    cnt_sh.at[pl.ds(sid * (_SEG // 16), _SEG // 16)])
    pltpu.sync_copy(fval.at[pl.ds(0, _AB // 16)],
                    a_sh.at[pl.ds(sid * (_AB // 16), _AB // 16)])
    pltpu.sync_copy(fval.at[pl.ds(0, _CB // 16)],
                    c_sh.at[pl.ds(sid * (_CB // 16), _CB // 16)])
    pltpu.sync_copy(x_h, xtab)
    pltpu.sync_copy(b_h, btab)
    plsc.subcore_barrier()

    for k in range(2):
        chunk = (1 - cid) if k == 0 else cid
        base = chunk * (_E // 2) + sid * _EPT
        pltpu.sync_copy(ei_h.at[pl.ds(_E + base, _EPT)], e_i1)
        pltpu.sync_copy(ea_h.at[pl.ds(base, _EPT)], e_i2)

        def seg_i(i):
            sl = pl.ds(i * 16, 16)
            segb[sl] = e_i1[sl] * 16 + e_i2[sl]
        _loop16(625, seg_i)
        pltpu.sync_copy(fones, cnt_sh.at[segb], add=True)

    nb = wid * _NPT
    lim = jnp.minimum(nb + _NPT, _N)

    def node_i(i):
        idx16 = nb + i * 16 + iota
        ok = idx16 < lim
        cidx = jnp.minimum(idx16, _N - 1)
        g16 = plsc.load_gather(btab, [cidx])
        v16 = plsc.load_gather(xtab, [cidx])
        nbuck[pl.ds(i * 16, 16)] = g16 * _V + v16
        nval[pl.ds(i * 16, 16)] = jnp.where(ok, 1.0, 0.0)
    _loop16(20, node_i, unroll=4)
    pltpu.sync_copy(nval, c_sh.at[nbuck], add=True)
    plsc.subcore_barrier()

    base = wid * _EPT
    pltpu.sync_copy(ei_h.at[pl.ds(base, _EPT)], e_i3)
    pltpu.sync_copy(cnt_sh.at[segb], fval)

    def buck_i(i):
        sl = pl.ds(i * 16, 16)
        g16 = plsc.load_gather(btab, [e_i1[sl]])
        v16 = plsc.load_gather(xtab, [e_i3[sl]])
        segb[sl] = e_i2[sl] * _CB + g16 * _V + v16
        fval[sl] = 1.0 / fval[sl]
    _loop16(625, buck_i)
    pltpu.sync_copy(fval, a_sh.at[segb], add=True)
    plsc.subcore_barrier()

    pltpu.sync_copy(a_sh.at[pl.ds(sid * (_AB // 16), _AB // 16)],
                    fval.at[pl.ds(0, _AB // 16)])
    pltpu.sync_copy(fval.at[pl.ds(0, _AB // 16)],
                    out_a.at[pl.ds(cid * _AB + sid * (_AB // 16), _AB // 16)])
    pltpu.sync_copy(c_sh.at[pl.ds(sid * (_CB // 16), _CB // 16)],
                    fones.at[pl.ds(0, _CB // 16)])
    pltpu.sync_copy(fones.at[pl.ds(0, _CB // 16)],
                    out_c.at[pl.ds(cid * _CB + sid * (_CB // 16), _CB // 16)])


@functools.lru_cache(maxsize=1)
def _get_sc_call():
    return pl.kernel(
        _sc_body,
        out_type=(jax.ShapeDtypeStruct((2 * _AB,), jnp.float32),
                  jax.ShapeDtypeStruct((2 * _CB,), jnp.float32)),
        mesh=plsc.VectorSubcoreMesh(core_axis_name="c", subcore_axis_name="s"),
        compiler_params=pltpu.CompilerParams(needs_layout_passes=False),
        scratch_types=[
            pltpu.VMEM((10000,), jnp.int32),
            pltpu.VMEM((10000,), jnp.int32),
            pltpu.VMEM((10000,), jnp.int32),
            pltpu.VMEM((10000,), jnp.int32),
            pltpu.VMEM((10000,), jnp.float32),
            pltpu.VMEM((10000,), jnp.float32),
            pltpu.VMEM((10000,), jnp.int32),
            pltpu.VMEM((10000,), jnp.int32),
            pltpu.VMEM((320,), jnp.int32),
            pltpu.VMEM((320,), jnp.float32),
            pltpu.VMEM_SHARED((_SEG,), jnp.float32),
            pltpu.VMEM_SHARED((_AB,), jnp.float32),
            pltpu.VMEM_SHARED((_CB,), jnp.float32),
        ],
    )


def _tc_body(a_ref, c_ref, emb_ref, wrel_ref, wroot_ref, bias_ref, out_ref):
    hi = jax.lax.Precision.HIGHEST
    emb = emb_ref[...]
    c0 = c_ref[0] + c_ref[1]
    acc = jnp.zeros((_G, _DO), jnp.float32)
    for r in range(_R):
        t_r = jax.lax.dot(emb, wrel_ref[r], precision=hi)
        a_r = a_ref[0, r] + a_ref[1, r]
        acc = acc + jax.lax.dot(a_r, t_r, precision=hi)
    t_root = jax.lax.dot(emb, wroot_ref[...], precision=hi)
    acc = acc + jax.lax.dot(c0, t_root, precision=hi)
    cg = jnp.sum(c0, axis=1, keepdims=True)
    acc = acc + cg * bias_ref[...]
    out_ref[...] = acc / jnp.maximum(cg, 1.0)


_tc_call = pl.pallas_call(
    _tc_body,
    out_shape=jax.ShapeDtypeStruct((_G, _DO), jnp.float32),
)


@jax.jit
def kernel(x, edge_index, edge_attr, batch, emb_table, W_rel, W_root, bias):
    ei = edge_index.reshape(-1).astype(jnp.int32)
    ea = edge_attr.reshape(-1).astype(jnp.int32)
    xf = x.reshape(-1).astype(jnp.int32)
    bt = batch.astype(jnp.int32)
    out_a, out_c = _get_sc_call()(ei, ea, xf, bt)
    a4 = out_a.reshape(2, _R, _G, _V)
    c3 = out_c.reshape(2, _G, _V)
    return _tc_call(a4, c3, emb_table, W_rel, W_root, bias.reshape(1, _DO))


# speedup vs baseline: 58.8308x; 1.0893x over previous
import functools

import jax
import jax.numpy as jnp
from jax import lax
from jax.experimental import pallas as pl
from jax.experimental.pallas import tpu as pltpu
from jax.experimental.pallas import tpu_sc as plsc

_N = 10000
_E = 320000
_G = 256
_R = 16
_V = 33
_DO = 64
_SEG = _N * _R
_AB = _R * _G * _V
_CB = _G * _V
_EPT = _E // 32
_NPT = 313


def _loop16(n, body, unroll=8):
    def b(i, c):
        body(i)
        return c
    lax.fori_loop(0, n, b, 0, unroll=unroll)


def _sc_body(seg_h, src_h, x_h, b_h, out_a, out_c,
             segb, sego, e_src, fones, fval, xtab, btab, nbuck, nval,
             cnt_sh, a_sh, c_sh):
    cid = lax.axis_index("c")
    sid = lax.axis_index("s")
    wid = cid * 16 + sid
    iota = lax.iota(jnp.int32, 16)

    def init_i(i):
        fones[pl.ds(i * 16, 16)] = jnp.full((16,), 1.0, jnp.float32)
        fval[pl.ds(i * 16, 16)] = jnp.zeros((16,), jnp.float32)
    _loop16(625, init_i)

    pltpu.sync_copy(fval.at[pl.ds(0, _SEG // 16)],
                    cnt_sh.at[pl.ds(sid * (_SEG // 16), _SEG // 16)])
    pltpu.sync_copy(fval.at[pl.ds(0, _AB // 16)],
                    a_sh.at[pl.ds(sid * (_AB // 16), _AB // 16)])
    pltpu.sync_copy(fval.at[pl.ds(0, _CB // 16)],
                    c_sh.at[pl.ds(sid * (_CB // 16), _CB // 16)])
    pltpu.sync_copy(x_h, xtab)
    pltpu.sync_copy(b_h, btab)
    plsc.subcore_barrier()

    other = sid + 16 * (1 - cid)
    pltpu.sync_copy(seg_h.at[pl.ds(other * _EPT, _EPT)], sego)
    pltpu.sync_copy(seg_h.at[pl.ds(wid * _EPT, _EPT)], segb)
    pltpu.sync_copy(fones, cnt_sh.at[sego], add=True)
    pltpu.sync_copy(fones, cnt_sh.at[segb], add=True)

    nb = wid * _NPT
    lim = jnp.minimum(nb + _NPT, _N)

    def node_i(i):
        idx16 = nb + i * 16 + iota
        ok = idx16 < lim
        cidx = jnp.minimum(idx16, _N - 1)
        g16 = plsc.load_gather(btab, [cidx])
        v16 = plsc.load_gather(xtab, [cidx])
        nbuck[pl.ds(i * 16, 16)] = v16 * _G + g16
        nval[pl.ds(i * 16, 16)] = jnp.where(ok, 1.0, 0.0)
    _loop16(20, node_i, unroll=4)
    pltpu.sync_copy(nval, c_sh.at[nbuck], add=True)
    plsc.subcore_barrier()

    pltpu.sync_copy(src_h.at[pl.ds(wid * _EPT, _EPT)], e_src)
    pltpu.sync_copy(cnt_sh.at[segb], fval)

    def buck_i(i):
        sl = pl.ds(i * 16, 16)
        s16 = segb[sl]
        g16 = plsc.load_gather(btab, [s16 >> 4])
        v16 = plsc.load_gather(xtab, [e_src[sl]])
        segb[sl] = (s16 & 15) * _CB + v16 * _G + g16
        fval[sl] = 1.0 / fval[sl]
    _loop16(625, buck_i)
    pltpu.sync_copy(fval, a_sh.at[segb], add=True)
    plsc.subcore_barrier()

    pltpu.sync_copy(a_sh.at[pl.ds(sid * (_AB // 16), _AB // 16)],
                    fval.at[pl.ds(0, _AB // 16)])
    pltpu.sync_copy(fval.at[pl.ds(0, _AB // 16)],
                    out_a.at[pl.ds(cid * _AB + sid * (_AB // 16), _AB // 16)])
    pltpu.sync_copy(c_sh.at[pl.ds(sid * (_CB // 16), _CB // 16)],
                    fones.at[pl.ds(0, _CB // 16)])
    pltpu.sync_copy(fones.at[pl.ds(0, _CB // 16)],
                    out_c.at[pl.ds(cid * _CB + sid * (_CB // 16), _CB // 16)])


@functools.lru_cache(maxsize=1)
def _get_sc_call():
    return pl.kernel(
        _sc_body,
        out_type=(jax.ShapeDtypeStruct((2 * _AB,), jnp.float32),
                  jax.ShapeDtypeStruct((2 * _CB,), jnp.float32)),
        mesh=plsc.VectorSubcoreMesh(core_axis_name="c", subcore_axis_name="s"),
        compiler_params=pltpu.CompilerParams(needs_layout_passes=False),
        scratch_types=[
            pltpu.VMEM((10000,), jnp.int32),
            pltpu.VMEM((10000,), jnp.int32),
            pltpu.VMEM((10000,), jnp.int32),
            pltpu.VMEM((10000,), jnp.float32),
            pltpu.VMEM((10000,), jnp.float32),
            pltpu.VMEM((10000,), jnp.int32),
            pltpu.VMEM((10000,), jnp.int32),
            pltpu.VMEM((320,), jnp.int32),
            pltpu.VMEM((320,), jnp.float32),
            pltpu.VMEM_SHARED((_SEG,), jnp.float32),
            pltpu.VMEM_SHARED((_AB,), jnp.float32),
            pltpu.VMEM_SHARED((_CB,), jnp.float32),
        ],
    )


def _tc_body(a_ref, c_ref, emb_ref, wrel_ref, wroot_ref, bias_ref, out_ref):
    hi = jax.lax.Precision.HIGHEST
    dn = (((0,), (1,)), ((), ()))
    emb = emb_ref[...]
    c2 = (c_ref[pl.ds(0, _CB)] + c_ref[pl.ds(_CB, _CB)]).reshape(_V, _G)
    acc = jnp.zeros((_DO, _G), jnp.float32)
    for r in range(_R):
        t_r = jax.lax.dot_general(wrel_ref[r], emb, dn, precision=hi)
        a2 = (a_ref[pl.ds(r * _CB, _CB)]
              + a_ref[pl.ds(_AB + r * _CB, _CB)]).reshape(_V, _G)
        acc = acc + jax.lax.dot(t_r, a2, precision=hi)
    t_root = jax.lax.dot_general(wroot_ref[...], emb, dn, precision=hi)
    acc = acc + jax.lax.dot(t_root, c2, precision=hi)
    cg = jnp.sum(c2, axis=0, keepdims=True)
    acc = acc + bias_ref[...] * cg
    out_ref[...] = (acc / jnp.maximum(cg, 1.0)).T


_tc_call = pl.pallas_call(
    _tc_body,
    out_shape=jax.ShapeDtypeStruct((_G, _DO), jnp.float32),
)


@jax.jit
def kernel(x, edge_index, edge_attr, batch, emb_table, W_rel, W_root, bias):
    seg = edge_index[1] * 16 + edge_attr.reshape(-1)
    src = edge_index[0]
    xf = x.reshape(-1)
    out_a, out_c = _get_sc_call()(seg, src, xf, batch)
    return _tc_call(out_a, out_c, emb_table, W_rel, W_root, bias.reshape(_DO, 1))


# speedup vs baseline: 68.4784x; 1.1640x over previous
import functools

import jax
import jax.numpy as jnp
from jax import lax
from jax.experimental import pallas as pl
from jax.experimental.pallas import tpu as pltpu
from jax.experimental.pallas import tpu_sc as plsc

_N = 10000
_E = 320000
_G = 256
_R = 16
_V = 33
_DO = 64
_SEG = _N * _R
_AB = _R * _G * _V
_CB = _G * _V
_EPT = _E // 32
_NPT = 313


def _loop16(n, body, unroll=8):
    def b(i, c):
        body(i)
        return c
    lax.fori_loop(0, n, b, 0, unroll=unroll)


def _sc_body(seg_h, ei_h, x_h, b_h, out_a, out_c,
             segb, sego, e_src, fones, fval, xtab, btab, nbuck, nval,
             cnt_sh, a_sh, c_sh):
    cid = lax.axis_index("c")
    sid = lax.axis_index("s")
    wid = cid * 16 + sid
    iota = lax.iota(jnp.int32, 16)

    def init_i(i):
        fones[pl.ds(i * 16, 16)] = jnp.full((16,), 1.0, jnp.float32)
        fval[pl.ds(i * 16, 16)] = jnp.zeros((16,), jnp.float32)
    _loop16(625, init_i)

    pltpu.sync_copy(fval.at[pl.ds(0, _SEG // 16)],
                    cnt_sh.at[pl.ds(sid * (_SEG // 16), _SEG // 16)])
    pltpu.sync_copy(fval.at[pl.ds(0, _AB // 16)],
                    a_sh.at[pl.ds(sid * (_AB // 16), _AB // 16)])
    pltpu.sync_copy(fval.at[pl.ds(0, _CB // 16)],
                    c_sh.at[pl.ds(sid * (_CB // 16), _CB // 16)])
    pltpu.sync_copy(x_h, xtab)
    pltpu.sync_copy(b_h, btab)
    plsc.subcore_barrier()

    other = sid + 16 * (1 - cid)
    pltpu.sync_copy(seg_h.at[pl.ds(other * _EPT, _EPT)], sego)
    pltpu.sync_copy(seg_h.at[pl.ds(wid * _EPT, _EPT)], segb)
    pltpu.sync_copy(fones, cnt_sh.at[sego], add=True)
    pltpu.sync_copy(fones, cnt_sh.at[segb], add=True)

    nb = wid * _NPT
    lim = jnp.minimum(nb + _NPT, _N)

    def node_i(i):
        idx16 = nb + i * 16 + iota
        ok = idx16 < lim
        cidx = jnp.minimum(idx16, _N - 1)
        g16 = plsc.load_gather(btab, [cidx])
        v16 = plsc.load_gather(xtab, [cidx])
        nbuck[pl.ds(i * 16, 16)] = v16 * _G + g16
        nval[pl.ds(i * 16, 16)] = jnp.where(ok, 1.0, 0.0)
    _loop16(20, node_i, unroll=4)
    pltpu.sync_copy(nval, c_sh.at[nbuck], add=True)
    plsc.subcore_barrier()

    pltpu.sync_copy(ei_h.at[pl.ds(wid * _EPT, _EPT)], e_src)
    pltpu.sync_copy(cnt_sh.at[segb], fval)

    def buck_i(i):
        sl = pl.ds(i * 16, 16)
        s16 = segb[sl]
        g16 = plsc.load_gather(btab, [s16 >> 4])
        v16 = plsc.load_gather(xtab, [e_src[sl]])
        segb[sl] = (s16 & 15) * _CB + v16 * _G + g16
        fval[sl] = 1.0 / fval[sl]
    _loop16(625, buck_i)
    pltpu.sync_copy(fval, a_sh.at[segb], add=True)
    plsc.subcore_barrier()

    pltpu.sync_copy(a_sh.at[pl.ds(sid * (_AB // 16), _AB // 16)],
                    fval.at[pl.ds(0, _AB // 16)])
    pltpu.sync_copy(fval.at[pl.ds(0, _AB // 16)],
                    out_a.at[pl.ds(cid * _AB + sid * (_AB // 16), _AB // 16)])
    pltpu.sync_copy(c_sh.at[pl.ds(sid * (_CB // 16), _CB // 16)],
                    fones.at[pl.ds(0, _CB // 16)])
    pltpu.sync_copy(fones.at[pl.ds(0, _CB // 16)],
                    out_c.at[pl.ds(cid * _CB + sid * (_CB // 16), _CB // 16)])


@functools.lru_cache(maxsize=1)
def _get_sc_call():
    return pl.kernel(
        _sc_body,
        out_type=(jax.ShapeDtypeStruct((2 * _AB,), jnp.float32),
                  jax.ShapeDtypeStruct((2 * _CB,), jnp.float32)),
        mesh=plsc.VectorSubcoreMesh(core_axis_name="c", subcore_axis_name="s"),
        compiler_params=pltpu.CompilerParams(needs_layout_passes=False),
        scratch_types=[
            pltpu.VMEM((10000,), jnp.int32),
            pltpu.VMEM((10000,), jnp.int32),
            pltpu.VMEM((10000,), jnp.int32),
            pltpu.VMEM((10000,), jnp.float32),
            pltpu.VMEM((10000,), jnp.float32),
            pltpu.VMEM((10000,), jnp.int32),
            pltpu.VMEM((10000,), jnp.int32),
            pltpu.VMEM((320,), jnp.int32),
            pltpu.VMEM((320,), jnp.float32),
            pltpu.VMEM_SHARED((_SEG,), jnp.float32),
            pltpu.VMEM_SHARED((_AB,), jnp.float32),
            pltpu.VMEM_SHARED((_CB,), jnp.float32),
        ],
    )


def _tc_body(a_ref, c_ref, emb_ref, wrel_ref, wroot_ref, bias_ref, out_ref):
    hi = jax.lax.Precision.HIGHEST
    dn = (((0,), (1,)), ((), ()))
    emb = emb_ref[...]
    c2 = (c_ref[pl.ds(0, _CB)] + c_ref[pl.ds(_CB, _CB)]).reshape(_V, _G)
    acc = jnp.zeros((_DO, _G), jnp.float32)
    for r in range(_R):
        t_r = jax.lax.dot_general(wrel_ref[r], emb, dn, precision=hi)
        a2 = (a_ref[pl.ds(r * _CB, _CB)]
              + a_ref[pl.ds(_AB + r * _CB, _CB)]).reshape(_V, _G)
        acc = acc + jax.lax.dot(t_r, a2, precision=hi)
    t_root = jax.lax.dot_general(wroot_ref[...], emb, dn, precision=hi)
    acc = acc + jax.lax.dot(t_root, c2, precision=hi)
    cg = jnp.sum(c2, axis=0, keepdims=True)
    acc = acc + bias_ref[...] * cg
    out_ref[...] = (acc / jnp.maximum(cg, 1.0)).T


_tc_call = pl.pallas_call(
    _tc_body,
    out_shape=jax.ShapeDtypeStruct((_G, _DO), jnp.float32),
)


@jax.jit
def kernel(x, edge_index, edge_attr, batch, emb_table, W_rel, W_root, bias):
    ei = edge_index.reshape(-1)
    seg = ei[_E:] * 16 + edge_attr.reshape(-1)
    xf = x.reshape(-1)
    out_a, out_c = _get_sc_call()(seg, ei, xf, batch)
    return _tc_call(out_a, out_c, emb_table, W_rel, W_root, bias.reshape(_DO, 1))


# speedup vs baseline: 72.7033x; 1.0617x over previous
import functools

import jax
import jax.numpy as jnp
from jax import lax
from jax.experimental import pallas as pl
from jax.experimental.pallas import tpu as pltpu
from jax.experimental.pallas import tpu_sc as plsc

_N = 10000
_E = 320000
_G = 256
_R = 16
_V = 33
_DO = 64
_SEG = _N * _R
_AB = _R * _G * _V
_CB = _G * _V
_EPT = _E // 32
_NPT = 313


def _loop16(n, body, unroll=8):
    def b(i, c):
        body(i)
        return c
    lax.fori_loop(0, n, b, 0, unroll=unroll)


def _sc_body(seg_h, ei_h, x_h, b_h, out_a, out_c,
             segb, sego, e_src, fones, fval, xtab, btab, nbuck, nval,
             sem_x, sem_b, sem_so, sem_sb, sem_sr,
             cnt_sh, a_sh, c_sh):
    cid = lax.axis_index("c")
    sid = lax.axis_index("s")
    wid = cid * 16 + sid
    iota = lax.iota(jnp.int32, 16)
    other = sid + 16 * (1 - cid)

    cp_x = pltpu.async_copy(x_h, xtab, sem_x)
    cp_b = pltpu.async_copy(b_h, btab, sem_b)
    cp_so = pltpu.async_copy(seg_h.at[pl.ds(other * _EPT, _EPT)], sego, sem_so)
    cp_sb = pltpu.async_copy(seg_h.at[pl.ds(wid * _EPT, _EPT)], segb, sem_sb)
    cp_sr = pltpu.async_copy(ei_h.at[pl.ds(wid * _EPT, _EPT)], e_src, sem_sr)

    def init_i(i):
        fones[pl.ds(i * 16, 16)] = jnp.full((16,), 1.0, jnp.float32)
        fval[pl.ds(i * 16, 16)] = jnp.zeros((16,), jnp.float32)
    _loop16(625, init_i)

    pltpu.sync_copy(fval.at[pl.ds(0, _SEG // 16)],
                    cnt_sh.at[pl.ds(sid * (_SEG // 16), _SEG // 16)])
    pltpu.sync_copy(fval.at[pl.ds(0, _AB // 16)],
                    a_sh.at[pl.ds(sid * (_AB // 16), _AB // 16)])
    pltpu.sync_copy(fval.at[pl.ds(0, _CB // 16)],
                    c_sh.at[pl.ds(sid * (_CB // 16), _CB // 16)])
    plsc.subcore_barrier()

    cp_so.wait()
    pltpu.sync_copy(fones, cnt_sh.at[sego], add=True)
    cp_sb.wait()
    pltpu.sync_copy(fones, cnt_sh.at[segb], add=True)
    cp_x.wait()
    cp_b.wait()

    nb = wid * _NPT
    lim = jnp.minimum(nb + _NPT, _N)

    def node_i(i):
        idx16 = nb + i * 16 + iota
        ok = idx16 < lim
        cidx = jnp.minimum(idx16, _N - 1)
        g16 = plsc.load_gather(btab, [cidx])
        v16 = plsc.load_gather(xtab, [cidx])
        nbuck[pl.ds(i * 16, 16)] = v16 * _G + g16
        nval[pl.ds(i * 16, 16)] = jnp.where(ok, 1.0, 0.0)
    _loop16(20, node_i, unroll=4)
    pltpu.sync_copy(nval, c_sh.at[nbuck], add=True)
    plsc.subcore_barrier()

    cp_sr.wait()
    pltpu.sync_copy(cnt_sh.at[segb], fval)

    def buck_i(i):
        sl = pl.ds(i * 16, 16)
        s16 = segb[sl]
        g16 = plsc.load_gather(btab, [s16 >> 4])
        v16 = plsc.load_gather(xtab, [e_src[sl]])
        segb[sl] = (s16 & 15) * _CB + v16 * _G + g16
        fval[sl] = 1.0 / fval[sl]
    _loop16(625, buck_i)
    pltpu.sync_copy(fval, a_sh.at[segb], add=True)
    plsc.subcore_barrier()

    pltpu.sync_copy(a_sh.at[pl.ds(sid * (_AB // 16), _AB // 16)],
                    fval.at[pl.ds(0, _AB // 16)])
    pltpu.sync_copy(fval.at[pl.ds(0, _AB // 16)],
                    out_a.at[pl.ds(cid * _AB + sid * (_AB // 16), _AB // 16)])
    pltpu.sync_copy(c_sh.at[pl.ds(sid * (_CB // 16), _CB // 16)],
                    fones.at[pl.ds(0, _CB // 16)])
    pltpu.sync_copy(fones.at[pl.ds(0, _CB // 16)],
                    out_c.at[pl.ds(cid * _CB + sid * (_CB // 16), _CB // 16)])


@functools.lru_cache(maxsize=1)
def _get_sc_call():
    return pl.kernel(
        _sc_body,
        out_type=(jax.ShapeDtypeStruct((2 * _AB,), jnp.float32),
                  jax.ShapeDtypeStruct((2 * _CB,), jnp.float32)),
        mesh=plsc.VectorSubcoreMesh(core_axis_name="c", subcore_axis_name="s"),
        compiler_params=pltpu.CompilerParams(needs_layout_passes=False),
        scratch_types=[
            pltpu.VMEM((10000,), jnp.int32),
            pltpu.VMEM((10000,), jnp.int32),
            pltpu.VMEM((10000,), jnp.int32),
            pltpu.VMEM((10000,), jnp.float32),
            pltpu.VMEM((10000,), jnp.float32),
            pltpu.VMEM((10000,), jnp.int32),
            pltpu.VMEM((10000,), jnp.int32),
            pltpu.VMEM((320,), jnp.int32),
            pltpu.VMEM((320,), jnp.float32),
            pltpu.SemaphoreType.DMA,
            pltpu.SemaphoreType.DMA,
            pltpu.SemaphoreType.DMA,
            pltpu.SemaphoreType.DMA,
            pltpu.SemaphoreType.DMA,
            pltpu.VMEM_SHARED((_SEG,), jnp.float32),
            pltpu.VMEM_SHARED((_AB,), jnp.float32),
            pltpu.VMEM_SHARED((_CB,), jnp.float32),
        ],
    )


def _tc_body(a_ref, c_ref, emb_ref, wrel_ref, wroot_ref, bias_ref, out_ref):
    hi = jax.lax.Precision.HIGHEST
    dn = (((0,), (1,)), ((), ()))
    emb = emb_ref[...]
    c2 = (c_ref[pl.ds(0, _CB)] + c_ref[pl.ds(_CB, _CB)]).reshape(_V, _G)
    acc = jnp.zeros((_DO, _G), jnp.float32)
    for r in range(_R):
        t_r = jax.lax.dot_general(wrel_ref[r], emb, dn, precision=hi)
        a2 = (a_ref[pl.ds(r * _CB, _CB)]
              + a_ref[pl.ds(_AB + r * _CB, _CB)]).reshape(_V, _G)
        acc = acc + jax.lax.dot(t_r, a2, precision=hi)
    t_root = jax.lax.dot_general(wroot_ref[...], emb, dn, precision=hi)
    acc = acc + jax.lax.dot(t_root, c2, precision=hi)
    cg = jnp.sum(c2, axis=0, keepdims=True)
    acc = acc + bias_ref[...] * cg
    out_ref[...] = (acc / jnp.maximum(cg, 1.0)).T


_tc_call = pl.pallas_call(
    _tc_body,
    out_shape=jax.ShapeDtypeStruct((_G, _DO), jnp.float32),
)


@jax.jit
def kernel(x, edge_index, edge_attr, batch, emb_table, W_rel, W_root, bias):
    ei = edge_index.reshape(-1)
    seg = ei[_E:] * 16 + edge_attr.reshape(-1)
    xf = x.reshape(-1)
    out_a, out_c = _get_sc_call()(seg, ei, xf, batch)
    return _tc_call(out_a, out_c, emb_table, W_rel, W_root, bias.reshape(_DO, 1))
